# Initial kernel scaffold; baseline (speedup 1.0000x reference)
import jax
import jax.numpy as jnp
from jax.experimental import pallas as pl


def kernel(x, edge_index, edge_index_pred, Wg1, bg1, Wg2, bg2, W1, b1, Wa1, W2, b2, Wa2, W3, b3, Wa3, Wf, bf):
    raise NotImplementedError("write your pallas kernel here")


# speedup vs baseline: 3.1313x; 3.1313x over previous
import functools

import jax
import jax.numpy as jnp
from jax import lax
from jax.experimental import pallas as pl
from jax.experimental.pallas import tpu as pltpu
from jax.experimental.pallas import tpu_sc as plsc

N = 10000
E = 320000
D = 128


def _tables_body(x_ref, ws_ref, wd_ref, bs_ref, t1_ref, t2_ref):
    x = x_ref[...]
    t1_ref[...] = jnp.dot(x, ws_ref[...], preferred_element_type=jnp.float32) + bs_ref[...]
    t2_ref[...] = jnp.dot(x, wd_ref[...], preferred_element_type=jnp.float32)


def _make_tables(x, ws, wd, bs):
    return pl.pallas_call(
        _tables_body,
        out_shape=(
            jax.ShapeDtypeStruct((N, 2 * 64), jnp.float32),
            jax.ShapeDtypeStruct((N, 2 * 64), jnp.float32),
        ),
    )(x, ws, wd, bs)


_NC = 2
_NS = 16
_NW = _NC * _NS
_EW = E // _NW
_CH = 200
_NCHUNK = _EW // _CH


def _sc_body(t1_hbm, t2_hbm, src_hbm, dst_hbm, out_hbm,
             idx1, idx2, g1, g2, zbuf, sem1, sem2):
    wid = lax.axis_index("s") * _NC + lax.axis_index("c")
    base = wid * _EW

    def chunk(c, carry):
        off = base + c * _CH
        pltpu.sync_copy(src_hbm.at[pl.ds(off, _CH)], idx1)
        pltpu.sync_copy(dst_hbm.at[pl.ds(off, _CH)], idx2)
        cp1 = pltpu.async_copy(t1_hbm.at[idx1], g1, sem1)
        cp2 = pltpu.async_copy(t2_hbm.at[idx2], g2, sem2)
        cp1.wait()
        cp2.wait()

        def row(r, rc):
            for j in range(4):
                u = g1[r, pl.ds(j * 16, 16)] + g2[r, pl.ds(j * 16, 16)]
                v = g1[r, pl.ds(64 + j * 16, 16)] + g2[r, pl.ds(64 + j * 16, 16)]
                z = jnp.maximum(jnp.maximum(u, 0.0) + v, 0.0)
                zbuf[r, pl.ds(j * 16, 16)] = z
            return rc

        lax.fori_loop(0, _CH, row, 0)
        pltpu.sync_copy(zbuf, out_hbm.at[pl.ds(off, _CH)])
        return carry

    lax.fori_loop(0, _NCHUNK, chunk, 0)


def _edge_block1(t1, t2, src, dst):
    mesh = plsc.VectorSubcoreMesh(core_axis_name="c", subcore_axis_name="s")
    k = functools.partial(
        pl.kernel,
        mesh=mesh,
        out_type=jax.ShapeDtypeStruct((E, 64), jnp.float32),
        scratch_types=[
            pltpu.VMEM((_CH,), jnp.int32),
            pltpu.VMEM((_CH,), jnp.int32),
            pltpu.VMEM((_CH, 128), jnp.float32),
            pltpu.VMEM((_CH, 128), jnp.float32),
            pltpu.VMEM((_CH, 64), jnp.float32),
            pltpu.SemaphoreType.DMA,
            pltpu.SemaphoreType.DMA,
        ],
    )(_sc_body)
    return k(t1, t2, src, dst)


_BE = 4000


def _tail_body(z_ref, w2_ref, b2_ref, wa2_ref, w3_ref, b3_ref, wa3_ref,
               wf_ref, bf_ref, o_ref):
    z1 = z_ref[...]
    h2 = jnp.maximum(jnp.dot(z1, w2_ref[...], preferred_element_type=jnp.float32) + b2_ref[...], 0.0)
    z2 = jnp.maximum(h2 + jnp.dot(z1, wa2_ref[...], preferred_element_type=jnp.float32), 0.0)
    h3 = jnp.maximum(jnp.dot(z2, w3_ref[...], preferred_element_type=jnp.float32) + b3_ref[...], 0.0)
    z3 = jnp.maximum(h3 + jnp.dot(z2, wa3_ref[...], preferred_element_type=jnp.float32), 0.0)
    s = jnp.dot(z3, wf_ref[...], preferred_element_type=jnp.float32) + bf_ref[...]
    o_ref[...] = jax.nn.sigmoid(s)


def _mlp_tail(z1, w2, b2, wa2, w3, b3, wa3, wf, bf):
    full = lambda shape: pl.BlockSpec(shape, lambda i: (0,) * len(shape))
    return pl.pallas_call(
        _tail_body,
        grid=(E // _BE,),
        in_specs=[
            pl.BlockSpec((_BE, 64), lambda i: (i, 0)),
            full((64, 32)), full((1, 32)), full((64, 32)),
            full((32, 16)), full((1, 16)), full((32, 16)),
            full((16, 1)), full((1, 1)),
        ],
        out_specs=pl.BlockSpec((_BE, 1), lambda i: (i, 0)),
        out_shape=jax.ShapeDtypeStruct((E, 1), jnp.float32),
    )(z1, w2, b2, wa2, w3, b3, wa3, wf, bf)


def kernel(x, edge_index, edge_index_pred, Wg1, bg1, Wg2, bg2,
           W1, b1, Wa1, W2, b2, Wa2, W3, b3, Wa3, Wf, bf):
    ws = jnp.concatenate([W1[:D], Wa1[:D]], axis=1)
    wd = jnp.concatenate([W1[D:], Wa1[D:]], axis=1)
    bs = jnp.concatenate([b1, jnp.zeros_like(b1)]).reshape(1, 2 * 64)

    t1, t2 = _make_tables(x, ws, wd, bs)

    src = edge_index_pred[0].astype(jnp.int32)
    dst = edge_index_pred[1].astype(jnp.int32)
    z1 = _edge_block1(t1, t2, src, dst)

    return _mlp_tail(
        z1,
        W2, b2.reshape(1, 32), Wa2,
        W3, b3.reshape(1, 16), Wa3,
        Wf, bf.reshape(1, 1),
    )


# speedup vs baseline: 4.1754x; 1.3334x over previous
import functools

import jax
import jax.numpy as jnp
from jax import lax
from jax.experimental import pallas as pl
from jax.experimental.pallas import tpu as pltpu
from jax.experimental.pallas import tpu_sc as plsc

N = 10000
E = 320000
D = 128


def _tables_body(x_ref, ws_ref, wd_ref, bs_ref, t1_ref, t2_ref):
    x = x_ref[...]
    t1_ref[...] = jnp.dot(x, ws_ref[...], preferred_element_type=jnp.float32) + bs_ref[...]
    t2_ref[...] = jnp.dot(x, wd_ref[...], preferred_element_type=jnp.float32)


def _make_tables(x, ws, wd, bs):
    return pl.pallas_call(
        _tables_body,
        out_shape=(
            jax.ShapeDtypeStruct((N, 2 * 64), jnp.float32),
            jax.ShapeDtypeStruct((N, 2 * 64), jnp.float32),
        ),
    )(x, ws, wd, bs)


_NC = 2
_NS = 16
_NW = _NC * _NS
_EW = E // _NW
_CH = 80
_NCHUNK = _EW // _CH


def _sc_body(t1_hbm, t2_hbm, src_hbm, dst_hbm, out_hbm,
             idxs, idxd, g1a, g1b, g2a, g2b, za, zb,
             sg1a, sg1b, sg2a, sg2b, ssa, ssb):
    wid = lax.axis_index("s") * _NC + lax.axis_index("c")
    base = wid * _EW
    g1 = (g1a, g1b)
    g2 = (g2a, g2b)
    zbuf = (za, zb)
    sg1 = (sg1a, sg1b)
    sg2 = (sg2a, sg2b)
    ss = (ssa, ssb)

    def islice(ref, c):
        return ref.at[pl.ds(c * _CH, _CH)]

    def issue_gather(c, b):
        pltpu.async_copy(t1_hbm.at[islice(idxs, c)], g1[b], sg1[b])
        pltpu.async_copy(t2_hbm.at[islice(idxd, c)], g2[b], sg2[b])

    def wait_gather(c, b):
        pltpu.make_async_copy(t1_hbm.at[islice(idxs, c)], g1[b], sg1[b]).wait()
        pltpu.make_async_copy(t2_hbm.at[islice(idxd, c)], g2[b], sg2[b]).wait()

    def out_slice(c):
        return out_hbm.at[pl.ds(base + c * _CH, _CH)]

    def wait_store(c, b):
        pltpu.make_async_copy(zbuf[b], out_slice(c), ss[b]).wait()

    def compute(b):
        gb1, gb2, zb_ = g1[b], g2[b], zbuf[b]

        def row(r, rc):
            for j in range(4):
                u = gb1[r, pl.ds(j * 16, 16)] + gb2[r, pl.ds(j * 16, 16)]
                v = gb1[r, pl.ds(64 + j * 16, 16)] + gb2[r, pl.ds(64 + j * 16, 16)]
                z = jnp.maximum(jnp.maximum(u, 0.0) + v, 0.0)
                zb_[r, pl.ds(j * 16, 16)] = z
            return rc

        lax.fori_loop(0, _CH, row, 0)

    def chunk_step(c, b, issue2, wait1, wait_st):
        nb = 1 - b
        if wait_st:
            wait_store(c, b)
        compute(b)
        if issue2:
            issue_gather(c + 2, b)
        pltpu.async_copy(zbuf[b], out_slice(c), ss[b])
        if wait1:
            wait_gather(c + 1, nb)

    pltpu.sync_copy(src_hbm.at[pl.ds(base, _EW)], idxs)
    pltpu.sync_copy(dst_hbm.at[pl.ds(base, _EW)], idxd)
    issue_gather(0, 0)
    wait_gather(0, 0)
    issue_gather(1, 1)
    chunk_step(0, 0, True, True, False)
    chunk_step(1, 1, True, True, False)

    def pair(i, carry):
        chunk_step(2 * i, 0, True, True, True)
        chunk_step(2 * i + 1, 1, True, True, True)
        return carry

    lax.fori_loop(1, (_NCHUNK - 3) // 2, pair, 0)
    chunk_step(_NCHUNK - 3, 0, True, True, True)
    chunk_step(_NCHUNK - 2, 1, False, True, True)
    chunk_step(_NCHUNK - 1, 0, False, False, True)
    wait_store(_NCHUNK - 2, 1)
    wait_store(_NCHUNK - 1, 0)


def _edge_block1(t1, t2, src, dst):
    mesh = plsc.VectorSubcoreMesh(core_axis_name="c", subcore_axis_name="s")
    k = functools.partial(
        pl.kernel,
        mesh=mesh,
        out_type=jax.ShapeDtypeStruct((E, 64), jnp.float32),
        scratch_types=[
            pltpu.VMEM((_EW,), jnp.int32),
            pltpu.VMEM((_EW,), jnp.int32),
            pltpu.VMEM((_CH, 128), jnp.float32),
            pltpu.VMEM((_CH, 128), jnp.float32),
            pltpu.VMEM((_CH, 128), jnp.float32),
            pltpu.VMEM((_CH, 128), jnp.float32),
            pltpu.VMEM((_CH, 64), jnp.float32),
            pltpu.VMEM((_CH, 64), jnp.float32),
            pltpu.SemaphoreType.DMA,
            pltpu.SemaphoreType.DMA,
            pltpu.SemaphoreType.DMA,
            pltpu.SemaphoreType.DMA,
            pltpu.SemaphoreType.DMA,
            pltpu.SemaphoreType.DMA,
        ],
    )(_sc_body)
    return k(t1, t2, src, dst)


_BE = 4000


def _tail_body(z_ref, w2_ref, b2_ref, wa2_ref, w3_ref, b3_ref, wa3_ref,
               wf_ref, bf_ref, o_ref):
    z1 = z_ref[...]
    h2 = jnp.maximum(jnp.dot(z1, w2_ref[...], preferred_element_type=jnp.float32) + b2_ref[...], 0.0)
    z2 = jnp.maximum(h2 + jnp.dot(z1, wa2_ref[...], preferred_element_type=jnp.float32), 0.0)
    h3 = jnp.maximum(jnp.dot(z2, w3_ref[...], preferred_element_type=jnp.float32) + b3_ref[...], 0.0)
    z3 = jnp.maximum(h3 + jnp.dot(z2, wa3_ref[...], preferred_element_type=jnp.float32), 0.0)
    s = jnp.dot(z3, wf_ref[...], preferred_element_type=jnp.float32) + bf_ref[...]
    o_ref[...] = jax.nn.sigmoid(s)


def _mlp_tail(z1, w2, b2, wa2, w3, b3, wa3, wf, bf):
    full = lambda shape: pl.BlockSpec(shape, lambda i: (0,) * len(shape))
    return pl.pallas_call(
        _tail_body,
        grid=(E // _BE,),
        in_specs=[
            pl.BlockSpec((_BE, 64), lambda i: (i, 0)),
            full((64, 32)), full((1, 32)), full((64, 32)),
            full((32, 16)), full((1, 16)), full((32, 16)),
            full((16, 1)), full((1, 1)),
        ],
        out_specs=pl.BlockSpec((_BE, 1), lambda i: (i, 0)),
        out_shape=jax.ShapeDtypeStruct((E, 1), jnp.float32),
    )(z1, w2, b2, wa2, w3, b3, wa3, wf, bf)


def kernel(x, edge_index, edge_index_pred, Wg1, bg1, Wg2, bg2,
           W1, b1, Wa1, W2, b2, Wa2, W3, b3, Wa3, Wf, bf):
    ws = jnp.concatenate([W1[:D], Wa1[:D]], axis=1)
    wd = jnp.concatenate([W1[D:], Wa1[D:]], axis=1)
    bs = jnp.concatenate([b1, jnp.zeros_like(b1)]).reshape(1, 2 * 64)

    t1, t2 = _make_tables(x, ws, wd, bs)

    src = edge_index_pred[0].astype(jnp.int32)
    dst = edge_index_pred[1].astype(jnp.int32)
    z1 = _edge_block1(t1, t2, src, dst)

    return _mlp_tail(
        z1,
        W2, b2.reshape(1, 32), Wa2,
        W3, b3.reshape(1, 16), Wa3,
        Wf, bf.reshape(1, 1),
    )
